# baseline (speedup 1.0000x reference)
import functools

import jax
import jax.numpy as jnp
from jax import lax
from jax.experimental import pallas as pl
from jax.experimental.pallas import tpu as pltpu
from jax.experimental.pallas import tpu_sc as plsc

NC, NS = 2, 16
NW = NC * NS
BB = 128
PH = 56
L = 16


@functools.lru_cache(maxsize=None)
def _build(BATCH, HIST, D):
    blocks_per_w = BATCH // BB // NW
    PD = 2 * D

    mesh = plsc.VectorSubcoreMesh(
        core_axis_name="c", subcore_axis_name="s",
        num_cores=NC, num_subcores=NS)

    @functools.partial(
        pl.kernel,
        mesh=mesh,
        compiler_params=pltpu.CompilerParams(
            use_tc_tiling_on_sc=True, needs_layout_passes=False),
        out_type=jax.ShapeDtypeStruct((HIST, D, BATCH), jnp.float32),
        scratch_types=[
            pltpu.VMEM((BB, PH), jnp.int32),
            pltpu.VMEM((BB,), jnp.int32),
            pltpu.VMEM((BB,), jnp.int32),
            pltpu.VMEM((BB, PD), jnp.float32),
            pltpu.VMEM((D, BB), jnp.float32),
            pltpu.SemaphoreType.DMA,
            pltpu.SemaphoreType.DMA,
        ],
    )
    def gather_kernel(idx_hbm, table_hbm, out_hbm, idx_v, col_v, off_v,
                      stage_v, panel_v, gsem, osem):
        wid = lax.axis_index("s") * NC + lax.axis_index("c")

        def block_body(k, carry):
            b0 = (wid * blocks_per_w + k) * BB
            pltpu.sync_copy(idx_hbm.at[pl.ds(b0, BB)], idx_v)

            def h_body(h, carry2):
                hvec = jnp.full((L,), h, dtype=jnp.int32)
                for g in range(BB // L):
                    rows = lax.iota(jnp.int32, L) + g * L
                    tid = plsc.load_gather(idx_v, [rows, hvec])
                    col_v[pl.ds(g * L, L)] = tid >> 1
                    off_v[pl.ds(g * L, L)] = (tid & 1) * D
                pltpu.async_copy(
                    table_hbm.at[col_v], stage_v, gsem).wait()
                def d_body(d, carry3):
                    for g in range(BB // L):
                        rows = lax.iota(jnp.int32, L) + g * L
                        lanes = off_v[pl.ds(g * L, L)] + d
                        vals = plsc.load_gather(stage_v, [rows, lanes])
                        panel_v[d, pl.ds(g * L, L)] = vals
                    return carry3

                lax.fori_loop(0, D, d_body, 0)
                pltpu.async_copy(
                    panel_v, out_hbm.at[h, pl.ds(0, D), pl.ds(b0, BB)],
                    osem)
                pltpu.make_async_copy(
                    out_hbm.at[0, pl.ds(0, D), pl.ds(b0, BB)], panel_v,
                    osem).wait()
                return carry2

            lax.fori_loop(0, HIST, h_body, 0)
            return carry

        lax.fori_loop(0, blocks_per_w, block_body, 0)

    return gather_kernel


def kernel(token_ids, w):
    BATCH, HIST = token_ids.shape
    V, D = w.shape
    idx2 = jnp.pad(token_ids.astype(jnp.int32), ((0, 0), (0, PH - HIST)))
    w2 = w.reshape(V // 2, 2 * D)
    res = _build(BATCH, HIST, D)(idx2, w2)
    return jnp.transpose(res, (2, 0, 1))


# speedup vs baseline: 2.2676x; 2.2676x over previous
import functools

import jax
import jax.numpy as jnp
from jax import lax
from jax.experimental import pallas as pl
from jax.experimental.pallas import tpu as pltpu
from jax.experimental.pallas import tpu_sc as plsc

NC, NS = 2, 16
NW = NC * NS
NB = 8


@functools.lru_cache(maxsize=None)
def _build(BATCH, HIST, D):
    b_per_w = BATCH // NW
    n_chunks = b_per_w // NB
    assert n_chunks % 2 == 0 and n_chunks >= 6

    mesh = plsc.VectorSubcoreMesh(
        core_axis_name="c", subcore_axis_name="s",
        num_cores=NC, num_subcores=NS)

    @functools.partial(
        pl.kernel,
        mesh=mesh,
        compiler_params=pltpu.CompilerParams(use_tc_tiling_on_sc=False),
        out_type=jax.ShapeDtypeStruct((BATCH, HIST, D), jnp.float32),
        scratch_types=[
            pltpu.VMEM((2, NB, HIST), jnp.int32),
            pltpu.VMEM((2, NB, HIST, D), jnp.float32),
            pltpu.SemaphoreType.DMA((2,)),
            pltpu.SemaphoreType.DMA((2,)),
        ],
    )
    def gather_kernel(idx_hbm, table_hbm, out_hbm, idx_v, rows_v, gsem, osem):
        wid = lax.axis_index("s") * NC + lax.axis_index("c")
        b_base = wid * b_per_w

        def fire(c, s):
            b0 = b_base + c * NB
            pltpu.sync_copy(idx_hbm.at[pl.ds(b0, NB)], idx_v.at[s])
            for i in range(NB):
                pltpu.async_copy(
                    table_hbm.at[idx_v.at[s, i]],
                    rows_v.at[s, i],
                    gsem.at[s])

        def retire(c, s):
            b0 = b_base + c * NB
            for i in range(NB):
                pltpu.make_async_copy(
                    table_hbm.at[pl.ds(0, HIST)], rows_v.at[s, i],
                    gsem.at[s]).wait()
            pltpu.async_copy(
                rows_v.at[s], out_hbm.at[pl.ds(b0, NB)], osem.at[s])

        def drain_out(s):
            pltpu.make_async_copy(
                out_hbm.at[pl.ds(b_base, NB)], rows_v.at[s],
                osem.at[s]).wait()

        fire(0, 0)
        fire(1, 1)
        retire(0, 0)
        drain_out(0)
        fire(2, 0)
        retire(1, 1)

        def body(g, carry):
            c0 = 2 * g
            drain_out(1)
            fire(c0 + 1, 1)
            retire(c0, 0)
            drain_out(0)
            fire(c0 + 2, 0)
            retire(c0 + 1, 1)
            return carry

        lax.fori_loop(1, n_chunks // 2 - 1, body, 0)

        c0 = n_chunks - 2
        drain_out(1)
        fire(c0 + 1, 1)
        retire(c0, 0)
        retire(c0 + 1, 1)
        drain_out(0)
        drain_out(1)

    return gather_kernel


def kernel(token_ids, w):
    BATCH, HIST = token_ids.shape
    V, D = w.shape
    return _build(BATCH, HIST, D)(token_ids.astype(jnp.int32), w)
